# baseline (speedup 1.0000x reference)
import functools

import jax
import jax.numpy as jnp
from jax import lax
from jax.experimental import pallas as pl
from jax.experimental.pallas import tpu as pltpu
from jax.experimental.pallas import tpu_sc as plsc

N = 10000
E = 320000
NFEAT = 128
NHEADS = 8
DHEAD = 16
ALPHA = 0.2

NC = 2
NS = 16
NW = NC * NS
CHUNK = 40
NCHUNKS = E // CHUNK
NK = NCHUNKS // 32
N_PAD = 10112
ROWS_PER_TILE = N_PAD // NS


def _dense_body(x_ref, wc_ref, wb_ref, ms_ref, md_ref, ab_ref,
                wh_ref, ex_ref):
    wh = jnp.dot(x_ref[...], wc_ref[...],
                 preferred_element_type=jnp.float32) + wb_ref[...]
    wh_ref[...] = wh.astype(jnp.bfloat16)
    a_s = jnp.dot(wh, ms_ref[...], preferred_element_type=jnp.float32)
    a_d = jnp.dot(wh, md_ref[...], preferred_element_type=jnp.float32)
    a_d = a_d + ab_ref[...]
    bnd = (jnp.max(a_s, axis=0, keepdims=True)
           + jnp.max(a_d, axis=0, keepdims=True))
    bnd = jnp.maximum(bnd, ALPHA * bnd)
    ex_ref[...] = jnp.concatenate(
        [a_s, a_d, jnp.broadcast_to(bnd, (N, 16)),
         jnp.zeros((N, 80), jnp.float32)], axis=1)


def _sc_body(wh_hbm, as_hbm, ad_hbm, bnd_hbm, src_hbm, dst_hbm,
             num_out, den_out,
             num_sh, den_sh, src_v, dst_v, asr_v, adr_v, whb_v, whr_v, w_v,
             bnd_v, sem_i, sem_g, sem_s):
    c = lax.axis_index("c")
    s = lax.axis_index("s")
    wid = s * NC + c
    pltpu.sync_copy(bnd_hbm, bnd_v)
    bndv = bnd_v[...]

    def off_of(k):
        kc = jnp.minimum(k, NK - 1)
        return (kc * NW + wid) * CHUNK

    def issue_idx(k):
        q = lax.rem(k, 4)
        off = off_of(k)
        a = pltpu.async_copy(src_hbm.at[pl.ds(off, CHUNK)], src_v.at[q], sem_i)
        b = pltpu.async_copy(dst_hbm.at[pl.ds(off, CHUNK)], dst_v.at[q], sem_i)
        return a, b

    def wait_idx():
        pltpu.make_async_copy(src_hbm.at[pl.ds(0, CHUNK)], src_v.at[0],
                              sem_i).wait()
        pltpu.make_async_copy(dst_hbm.at[pl.ds(0, CHUNK)], dst_v.at[0],
                              sem_i).wait()

    def issue_gathers(k):
        q = lax.rem(k, 4)
        p = lax.rem(k, 2) * CHUNK
        pltpu.async_copy(as_hbm.at[src_v.at[q]], asr_v.at[pl.ds(p, CHUNK)],
                         sem_g)
        pltpu.async_copy(ad_hbm.at[dst_v.at[q]], adr_v.at[pl.ds(p, CHUNK)],
                         sem_g)
        pltpu.async_copy(wh_hbm.at[src_v.at[q]], whb_v.at[pl.ds(p, CHUNK)],
                         sem_g)

    def wait_gathers():
        pltpu.make_async_copy(as_hbm.at[src_v.at[0]],
                              asr_v.at[pl.ds(0, CHUNK)], sem_g).wait()
        pltpu.make_async_copy(ad_hbm.at[dst_v.at[0]],
                              adr_v.at[pl.ds(0, CHUNK)], sem_g).wait()
        pltpu.make_async_copy(wh_hbm.at[src_v.at[0]],
                              whb_v.at[pl.ds(0, CHUNK)], sem_g).wait()

    def issue_scatters(k):
        q = lax.rem(k, 4)
        p = lax.rem(k, 2) * CHUNK
        pltpu.async_copy(w_v.at[pl.ds(p, CHUNK)], den_sh.at[dst_v.at[q]],
                         sem_s, add=True)
        pltpu.async_copy(whr_v.at[pl.ds(p, CHUNK)], num_sh.at[dst_v.at[q]],
                         sem_s, add=True)

    def wait_scatters():
        pltpu.make_async_copy(w_v.at[pl.ds(0, CHUNK)], den_sh.at[dst_v.at[0]],
                              sem_s).wait()
        pltpu.make_async_copy(whr_v.at[pl.ds(0, CHUNK)],
                              num_sh.at[dst_v.at[0]], sem_s).wait()

    def compute(k):
        p = lax.rem(k, 2) * CHUNK

        @plsc.parallel_loop(0, CHUNK, 1, unroll=4)
        def _edge(i):
            r = p + i
            t = asr_v[r, :] + adr_v[r, :]
            w = jnp.exp(jnp.maximum(t, ALPHA * t) - bndv)
            w_v[r, :] = w
            for j in range(NHEADS):
                wh16 = whb_v[r, pl.ds(j * 16, 16)].astype(jnp.float32)
                whr_v[r, pl.ds(j * 16, 16)] = w * wh16

    issue_idx(0)
    issue_idx(1)

    zv = jnp.zeros((16,), jnp.float32)
    row0 = s * ROWS_PER_TILE

    @plsc.parallel_loop(0, 2 * CHUNK, 1, unroll=4)
    def _zwv(i):
        w_v[i, :] = zv

    @plsc.parallel_loop(0, CHUNK, 1, unroll=4)
    def _zwhr(i):
        for j in range(NHEADS):
            whr_v[CHUNK + i, pl.ds(j * 16, 16)] = zv

    for t in range(ROWS_PER_TILE // CHUNK):
        pltpu.sync_copy(whr_v.at[pl.ds(CHUNK, CHUNK)],
                        num_sh.at[pl.ds(row0 + t * CHUNK, CHUNK)])
    pltpu.sync_copy(whr_v.at[pl.ds(CHUNK, ROWS_PER_TILE % CHUNK)],
                    num_sh.at[pl.ds(row0 + ROWS_PER_TILE - ROWS_PER_TILE % CHUNK,
                                    ROWS_PER_TILE % CHUNK)])
    for t in range(ROWS_PER_TILE // (2 * CHUNK)):
        pltpu.sync_copy(w_v.at[pl.ds(0, 2 * CHUNK)],
                        den_sh.at[pl.ds(row0 + t * 2 * CHUNK, 2 * CHUNK)])
    pltpu.sync_copy(
        w_v.at[pl.ds(0, ROWS_PER_TILE % (2 * CHUNK))],
        den_sh.at[pl.ds(row0 + ROWS_PER_TILE - ROWS_PER_TILE % (2 * CHUNK),
                        ROWS_PER_TILE % (2 * CHUNK))])

    wait_idx()
    issue_gathers(0)
    plsc.subcore_barrier()

    def step(k, carry):
        wait_gathers()
        wait_idx()

        @pl.when(k > 0)
        def _():
            wait_scatters()

        issue_gathers(k + 1)
        compute(k)
        issue_scatters(k)
        issue_idx(k + 2)
        return carry

    lax.fori_loop(0, NK, step, 0)
    wait_gathers()
    wait_scatters()
    wait_idx()
    plsc.subcore_barrier()
    pltpu.sync_copy(num_sh.at[pl.ds(row0, ROWS_PER_TILE)],
                    num_out.at[c, pl.ds(row0, ROWS_PER_TILE)])
    pltpu.sync_copy(den_sh.at[pl.ds(row0, ROWS_PER_TILE)],
                    den_out.at[c, pl.ds(row0, ROWS_PER_TILE)])


def _combine_body(num_ref, den_ref, brep_ref, pmat_ref, out_ref):
    num = num_ref[0, :N, :] + num_ref[1, :N, :]
    den16 = den_ref[0, :N, :] + den_ref[1, :N, :]
    dex = jnp.dot(den16, brep_ref[...], preferred_element_type=jnp.float32)
    q = num / jnp.where(dex > 0, dex, 1.0)
    out_ref[...] = jnp.dot(q, pmat_ref[...], preferred_element_type=jnp.float32)


def kernel(x, edge_index, W, Wb, A, Ab):
    f32 = jnp.float32
    jcol = jnp.arange(128)
    perm = (jcol % NHEADS) * DHEAD + jcol // NHEADS
    Wc = W.transpose(1, 0, 2).reshape(NFEAT, NHEADS * DHEAD)[:, perm]
    Wb_c = Wb.reshape(1, NHEADS * DHEAD)[:, perm]
    colh = jnp.arange(16) % NHEADS
    rowh = jcol % NHEADS
    rowd = jcol // NHEADS
    sel = rowh[:, None] == colh[None, :]
    Ms = jnp.where(sel, A[colh[None, :], rowd[:, None]], 0.0).astype(f32)
    Md = jnp.where(sel, A[colh[None, :], 16 + rowd[:, None]], 0.0).astype(f32)
    ab_row = Ab[colh][None, :].astype(f32)

    wh, extras = pl.pallas_call(
        _dense_body,
        out_shape=[jax.ShapeDtypeStruct((N, 128), jnp.bfloat16),
                   jax.ShapeDtypeStruct((N, 128), f32)],
    )(x, Wc, Wb_c, Ms, Md, ab_row)
    atab_s = extras[:, 0:16]
    atab_d = extras[:, 16:32]
    bnd = extras[0, 32:48]

    src = edge_index[0]
    dst = edge_index[1]

    mesh = plsc.VectorSubcoreMesh(core_axis_name="c", subcore_axis_name="s",
                                  num_cores=NC, num_subcores=NS)
    sc = pl.kernel(
        _sc_body,
        out_type=[jax.ShapeDtypeStruct((NC, N_PAD, 128), f32),
                  jax.ShapeDtypeStruct((NC, N_PAD, 16), f32)],
        mesh=mesh,
        compiler_params=pltpu.CompilerParams(use_tc_tiling_on_sc=False),
        scratch_types=[
            pltpu.VMEM_SHARED((N_PAD, 128), f32),
            pltpu.VMEM_SHARED((N_PAD, 16), f32),
            pltpu.VMEM((4, CHUNK), jnp.int32),
            pltpu.VMEM((4, CHUNK), jnp.int32),
            pltpu.VMEM((2 * CHUNK, 16), f32),
            pltpu.VMEM((2 * CHUNK, 16), f32),
            pltpu.VMEM((2 * CHUNK, 128), jnp.bfloat16),
            pltpu.VMEM((2 * CHUNK, 128), f32),
            pltpu.VMEM((2 * CHUNK, 16), f32),
            pltpu.VMEM((16,), f32),
            pltpu.SemaphoreType.DMA,
            pltpu.SemaphoreType.DMA,
            pltpu.SemaphoreType.DMA,
        ],
    )
    num_parts, den_parts = sc(wh, atab_s, atab_d, bnd, src, dst)

    brep = (jnp.arange(16)[:, None] == (jcol[None, :] % NHEADS)).astype(f32)
    pj = (jcol % DHEAD) * NHEADS + jcol // DHEAD
    pmat = (jcol[:, None] == pj[None, :]).astype(f32)
    out = pl.pallas_call(
        _combine_body,
        out_shape=jax.ShapeDtypeStruct((N, 128), f32),
    )(num_parts, den_parts, brep, pmat)
    return out


# speedup vs baseline: 1.1424x; 1.1424x over previous
import jax
import jax.numpy as jnp
from jax import lax
from jax.experimental import pallas as pl
from jax.experimental.pallas import tpu as pltpu
from jax.experimental.pallas import tpu_sc as plsc

N = 10000
E = 320000
NFEAT = 128
NHEADS = 8
DHEAD = 16
ALPHA = 0.2

NC = 2
NS = 16
NW = NC * NS
CHUNK = 80
NCHUNKS = E // CHUNK
NK = NCHUNKS // NW
N_PAD = 10112
ROWS_PER_TILE = N_PAD // NS
WIDE = NHEADS * DHEAD + DHEAD


def _dense_body(x_ref, wc_ref, wb_ref, ms_ref, md_ref, ab_ref,
                whx_ref, ex_ref):
    wh = jnp.dot(x_ref[...], wc_ref[...],
                 preferred_element_type=jnp.float32) + wb_ref[...]
    a_s = jnp.dot(wh, ms_ref[...], preferred_element_type=jnp.float32)
    a_d = jnp.dot(wh, md_ref[...], preferred_element_type=jnp.float32)
    a_d = a_d + ab_ref[...]
    bnd = (jnp.max(a_s, axis=0, keepdims=True)
           + jnp.max(a_d, axis=0, keepdims=True))
    bnd = jnp.maximum(bnd, ALPHA * bnd)
    whx_ref[...] = jnp.concatenate([wh, a_s], axis=1)
    ex_ref[...] = jnp.concatenate(
        [a_d, jnp.broadcast_to(bnd, (N, 16)),
         jnp.zeros((N, 96), jnp.float32)], axis=1)


def _sc_body(whx_hbm, ad_hbm, bnd_hbm, src_hbm, dst_hbm,
             acc_out,
             acc_sh, src_v, dst_v, adr_v, whr_v, bnd_v,
             sem_i, sem_g, sem_s):
    c = lax.axis_index("c")
    s = lax.axis_index("s")
    wid = s * NC + c
    pltpu.sync_copy(bnd_hbm, bnd_v)
    bndv = bnd_v[...]

    def off_of(k):
        kc = jnp.minimum(k, NK - 1)
        return (kc * NW + wid) * CHUNK

    def issue_idx(k):
        q = lax.rem(k, 4)
        off = off_of(k)
        a = pltpu.async_copy(src_hbm.at[pl.ds(off, CHUNK)], src_v.at[q], sem_i)
        b = pltpu.async_copy(dst_hbm.at[pl.ds(off, CHUNK)], dst_v.at[q], sem_i)
        return a, b

    def wait_idx():
        pltpu.make_async_copy(src_hbm.at[pl.ds(0, CHUNK)], src_v.at[0],
                              sem_i).wait()
        pltpu.make_async_copy(dst_hbm.at[pl.ds(0, CHUNK)], dst_v.at[0],
                              sem_i).wait()

    def issue_gathers(k):
        q = lax.rem(k, 4)
        p = lax.rem(k, 2) * CHUNK
        pltpu.async_copy(ad_hbm.at[dst_v.at[q]], adr_v.at[pl.ds(p, CHUNK)],
                         sem_g)
        pltpu.async_copy(whx_hbm.at[src_v.at[q]], whr_v.at[pl.ds(p, CHUNK)],
                         sem_g)

    def wait_gathers():
        pltpu.make_async_copy(ad_hbm.at[dst_v.at[0]],
                              adr_v.at[pl.ds(0, CHUNK)], sem_g).wait()
        pltpu.make_async_copy(whx_hbm.at[src_v.at[0]],
                              whr_v.at[pl.ds(0, CHUNK)], sem_g).wait()

    def issue_scatters(k):
        q = lax.rem(k, 4)
        p = lax.rem(k, 2) * CHUNK
        pltpu.async_copy(whr_v.at[pl.ds(p, CHUNK)], acc_sh.at[dst_v.at[q]],
                         sem_s, add=True)

    def wait_scatters():
        pltpu.make_async_copy(whr_v.at[pl.ds(0, CHUNK)],
                              acc_sh.at[dst_v.at[0]], sem_s).wait()

    def compute(k):
        p = lax.rem(k, 2) * CHUNK

        @plsc.parallel_loop(0, CHUNK, 1, unroll=4)
        def _edge(i):
            r = p + i
            t = whr_v[r, pl.ds(128, 16)] + adr_v[r, :]
            w = jnp.exp(jnp.maximum(t, ALPHA * t) - bndv)
            whr_v[r, pl.ds(128, 16)] = w
            for j in range(NHEADS):
                whr_v[r, pl.ds(j * 16, 16)] = w * whr_v[r, pl.ds(j * 16, 16)]

    issue_idx(0)
    issue_idx(1)

    zv = jnp.zeros((16,), jnp.float32)
    row0 = s * ROWS_PER_TILE

    @plsc.parallel_loop(0, CHUNK, 1, unroll=4)
    def _zwhr(i):
        for j in range(WIDE // 16):
            whr_v[CHUNK + i, pl.ds(j * 16, 16)] = zv

    for t in range(ROWS_PER_TILE // CHUNK):
        pltpu.sync_copy(whr_v.at[pl.ds(CHUNK, CHUNK)],
                        acc_sh.at[pl.ds(row0 + t * CHUNK, CHUNK)])
    pltpu.sync_copy(whr_v.at[pl.ds(CHUNK, ROWS_PER_TILE % CHUNK)],
                    acc_sh.at[pl.ds(row0 + ROWS_PER_TILE - ROWS_PER_TILE % CHUNK,
                                    ROWS_PER_TILE % CHUNK)])

    wait_idx()
    issue_gathers(0)
    plsc.subcore_barrier()

    def step(k, carry):
        wait_gathers()
        wait_idx()

        @pl.when(k > 0)
        def _():
            wait_scatters()

        issue_gathers(k + 1)
        compute(k)
        issue_scatters(k)
        issue_idx(k + 2)
        return carry

    lax.fori_loop(0, NK, step, 0)
    wait_gathers()
    wait_scatters()
    wait_idx()
    plsc.subcore_barrier()
    pltpu.sync_copy(acc_sh.at[pl.ds(row0, ROWS_PER_TILE)],
                    acc_out.at[c, pl.ds(row0, ROWS_PER_TILE)])


def _combine_body(acc_ref, brep_ref, pmat_ref, out_ref):
    num = acc_ref[0, :N, 0:128] + acc_ref[1, :N, 0:128]
    den16 = acc_ref[0, :N, 128:WIDE] + acc_ref[1, :N, 128:WIDE]
    dex = jnp.dot(den16, brep_ref[...], preferred_element_type=jnp.float32)
    q = num / jnp.where(dex > 0, dex, 1.0)
    out_ref[...] = jnp.dot(q, pmat_ref[...], preferred_element_type=jnp.float32)


def kernel(x, edge_index, W, Wb, A, Ab):
    f32 = jnp.float32
    jcol = jnp.arange(128)
    perm = (jcol % NHEADS) * DHEAD + jcol // NHEADS
    Wc = W.transpose(1, 0, 2).reshape(NFEAT, NHEADS * DHEAD)[:, perm]
    Wb_c = Wb.reshape(1, NHEADS * DHEAD)[:, perm]
    colh = jnp.arange(16) % NHEADS
    rowh = jcol % NHEADS
    rowd = jcol // NHEADS
    sel = rowh[:, None] == colh[None, :]
    Ms = jnp.where(sel, A[colh[None, :], rowd[:, None]], 0.0).astype(f32)
    Md = jnp.where(sel, A[colh[None, :], 16 + rowd[:, None]], 0.0).astype(f32)
    ab_row = Ab[colh][None, :].astype(f32)

    whx, extras = pl.pallas_call(
        _dense_body,
        out_shape=[jax.ShapeDtypeStruct((N, WIDE), f32),
                   jax.ShapeDtypeStruct((N, 128), f32)],
    )(x, Wc, Wb_c, Ms, Md, ab_row)
    atab_d = extras[:, 0:16]
    bnd = extras[0, 16:32]

    src = edge_index[0]
    dst = edge_index[1]

    mesh = plsc.VectorSubcoreMesh(core_axis_name="c", subcore_axis_name="s",
                                  num_cores=NC, num_subcores=NS)
    sc = pl.kernel(
        _sc_body,
        out_type=jax.ShapeDtypeStruct((NC, N_PAD, WIDE), f32),
        mesh=mesh,
        compiler_params=pltpu.CompilerParams(use_tc_tiling_on_sc=False),
        scratch_types=[
            pltpu.VMEM_SHARED((N_PAD, WIDE), f32),
            pltpu.VMEM((4, CHUNK), jnp.int32),
            pltpu.VMEM((4, CHUNK), jnp.int32),
            pltpu.VMEM((2 * CHUNK, 16), f32),
            pltpu.VMEM((2 * CHUNK, WIDE), f32),
            pltpu.VMEM((16,), f32),
            pltpu.SemaphoreType.DMA,
            pltpu.SemaphoreType.DMA,
            pltpu.SemaphoreType.DMA,
        ],
    )
    acc_parts = sc(whx, atab_d, bnd, src, dst)

    brep = (jnp.arange(16)[:, None] == (jcol[None, :] % NHEADS)).astype(f32)
    pj = (jcol % DHEAD) * NHEADS + jcol // DHEAD
    pmat = (jcol[:, None] == pj[None, :]).astype(f32)
    out = pl.pallas_call(
        _combine_body,
        out_shape=jax.ShapeDtypeStruct((N, 128), f32),
    )(acc_parts, brep, pmat)
    return out


# speedup vs baseline: 1.2841x; 1.1241x over previous
import jax
import jax.numpy as jnp
from jax import lax
from jax.experimental import pallas as pl
from jax.experimental.pallas import tpu as pltpu
from jax.experimental.pallas import tpu_sc as plsc

N = 10000
E = 320000
NFEAT = 128
NHEADS = 8
DHEAD = 16
ALPHA = 0.2

NC = 2
NS = 16
NW = NC * NS
CHUNK = 80
NCHUNKS = E // CHUNK
NK = NCHUNKS // NW
N_PAD = 10112
ROWS_PER_TILE = N_PAD // NS


def _dense_body(x_ref, wc_ref, wb_ref, ms_ref, md_ref, ab_ref,
                wh_ref, ex_ref):
    wh = jnp.dot(x_ref[...], wc_ref[...],
                 preferred_element_type=jnp.float32) + wb_ref[...]
    wh_ref[...] = wh.astype(jnp.bfloat16)
    a_s = jnp.dot(wh, ms_ref[...], preferred_element_type=jnp.float32)
    a_d = jnp.dot(wh, md_ref[...], preferred_element_type=jnp.float32)
    a_d = a_d + ab_ref[...]
    bnd = (jnp.max(a_s, axis=0, keepdims=True)
           + jnp.max(a_d, axis=0, keepdims=True))
    bnd = jnp.maximum(bnd, ALPHA * bnd)
    ex_ref[...] = jnp.concatenate(
        [a_s, a_d, jnp.broadcast_to(bnd, (N, 16)),
         jnp.zeros((N, 80), jnp.float32)], axis=1)


def _sc_body(wh_hbm, as_hbm, ad_hbm, bnd_hbm, src_hbm, dst_hbm,
             num_out, den_out,
             num_sh, den_sh, src_v, dst_v, asr_v, adr_v, whb_v, whr_v,
             bnd_v, sem_i, sem_g, sem_s):
    c = lax.axis_index("c")
    s = lax.axis_index("s")
    wid = s * NC + c
    pltpu.sync_copy(bnd_hbm, bnd_v)
    bndv = bnd_v[...]

    def off_of(k):
        kc = jnp.minimum(k, NK - 1)
        return (kc * NW + wid) * CHUNK

    def issue_idx(k):
        q = lax.rem(k, 4)
        off = off_of(k)
        a = pltpu.async_copy(src_hbm.at[pl.ds(off, CHUNK)], src_v.at[q], sem_i)
        b = pltpu.async_copy(dst_hbm.at[pl.ds(off, CHUNK)], dst_v.at[q], sem_i)
        return a, b

    def wait_idx():
        pltpu.make_async_copy(src_hbm.at[pl.ds(0, CHUNK)], src_v.at[0],
                              sem_i).wait()
        pltpu.make_async_copy(dst_hbm.at[pl.ds(0, CHUNK)], dst_v.at[0],
                              sem_i).wait()

    def issue_gathers(k):
        q = lax.rem(k, 4)
        p = lax.rem(k, 2) * CHUNK
        pltpu.async_copy(as_hbm.at[src_v.at[q]], asr_v.at[pl.ds(p, CHUNK)],
                         sem_g)
        pltpu.async_copy(ad_hbm.at[dst_v.at[q]], adr_v.at[pl.ds(p, CHUNK)],
                         sem_g)
        pltpu.async_copy(wh_hbm.at[src_v.at[q]], whb_v.at[pl.ds(p, CHUNK)],
                         sem_g)

    def wait_gathers():
        pltpu.make_async_copy(as_hbm.at[src_v.at[0]],
                              asr_v.at[pl.ds(0, CHUNK)], sem_g).wait()
        pltpu.make_async_copy(ad_hbm.at[dst_v.at[0]],
                              adr_v.at[pl.ds(0, CHUNK)], sem_g).wait()
        pltpu.make_async_copy(wh_hbm.at[src_v.at[0]],
                              whb_v.at[pl.ds(0, CHUNK)], sem_g).wait()

    def issue_scatters(k):
        q = lax.rem(k, 4)
        p = lax.rem(k, 2) * CHUNK
        pltpu.async_copy(asr_v.at[pl.ds(p, CHUNK)], den_sh.at[dst_v.at[q]],
                         sem_s, add=True)
        pltpu.async_copy(whr_v.at[pl.ds(p, CHUNK)], num_sh.at[dst_v.at[q]],
                         sem_s, add=True)

    def wait_scatters():
        pltpu.make_async_copy(asr_v.at[pl.ds(0, CHUNK)],
                              den_sh.at[dst_v.at[0]], sem_s).wait()
        pltpu.make_async_copy(whr_v.at[pl.ds(0, CHUNK)],
                              num_sh.at[dst_v.at[0]], sem_s).wait()

    def compute(k):
        p = lax.rem(k, 2) * CHUNK

        @plsc.parallel_loop(0, CHUNK, 1, unroll=4)
        def _edge(i):
            r = p + i
            t = asr_v[r, :] + adr_v[r, :]
            w = jnp.exp(jnp.maximum(t, ALPHA * t) - bndv)
            asr_v[r, :] = w
            for j in range(NHEADS):
                wh16 = whb_v[r, pl.ds(j * 16, 16)].astype(jnp.float32)
                whr_v[r, pl.ds(j * 16, 16)] = w * wh16

    issue_idx(0)
    issue_idx(1)

    zv = jnp.zeros((16,), jnp.float32)
    row0 = s * ROWS_PER_TILE

    @plsc.parallel_loop(0, 2 * CHUNK, 1, unroll=4)
    def _zasr(i):
        asr_v[i, :] = zv

    @plsc.parallel_loop(0, CHUNK, 1, unroll=4)
    def _zwhr(i):
        for j in range(NHEADS):
            whr_v[CHUNK + i, pl.ds(j * 16, 16)] = zv

    for t in range(ROWS_PER_TILE // CHUNK):
        pltpu.sync_copy(whr_v.at[pl.ds(CHUNK, CHUNK)],
                        num_sh.at[pl.ds(row0 + t * CHUNK, CHUNK)])
    pltpu.sync_copy(whr_v.at[pl.ds(CHUNK, ROWS_PER_TILE % CHUNK)],
                    num_sh.at[pl.ds(row0 + ROWS_PER_TILE - ROWS_PER_TILE % CHUNK,
                                    ROWS_PER_TILE % CHUNK)])
    for t in range(ROWS_PER_TILE // (2 * CHUNK)):
        pltpu.sync_copy(asr_v.at[pl.ds(0, 2 * CHUNK)],
                        den_sh.at[pl.ds(row0 + t * 2 * CHUNK, 2 * CHUNK)])
    pltpu.sync_copy(
        asr_v.at[pl.ds(0, ROWS_PER_TILE % (2 * CHUNK))],
        den_sh.at[pl.ds(row0 + ROWS_PER_TILE - ROWS_PER_TILE % (2 * CHUNK),
                        ROWS_PER_TILE % (2 * CHUNK))])

    wait_idx()
    issue_gathers(0)
    plsc.subcore_barrier()

    def step(k, carry):
        wait_gathers()
        wait_idx()

        @pl.when(k > 0)
        def _():
            wait_scatters()

        issue_gathers(k + 1)
        compute(k)
        issue_scatters(k)
        issue_idx(k + 2)
        return carry

    lax.fori_loop(0, NK, step, 0)
    wait_gathers()
    wait_scatters()
    wait_idx()
    plsc.subcore_barrier()
    pltpu.sync_copy(num_sh.at[pl.ds(row0, ROWS_PER_TILE)],
                    num_out.at[c, pl.ds(row0, ROWS_PER_TILE)])
    pltpu.sync_copy(den_sh.at[pl.ds(row0, ROWS_PER_TILE)],
                    den_out.at[c, pl.ds(row0, ROWS_PER_TILE)])


def _combine_body(num_ref, den_ref, brep_ref, pmat_ref, out_ref):
    num = num_ref[0, :N, :] + num_ref[1, :N, :]
    den16 = den_ref[0, :N, :] + den_ref[1, :N, :]
    dex = jnp.dot(den16, brep_ref[...], preferred_element_type=jnp.float32)
    q = num / jnp.where(dex > 0, dex, 1.0)
    out_ref[...] = jnp.dot(q, pmat_ref[...], preferred_element_type=jnp.float32)


def kernel(x, edge_index, W, Wb, A, Ab):
    f32 = jnp.float32
    jcol = jnp.arange(128)
    perm = (jcol % NHEADS) * DHEAD + jcol // NHEADS
    Wc = W.transpose(1, 0, 2).reshape(NFEAT, NHEADS * DHEAD)[:, perm]
    Wb_c = Wb.reshape(1, NHEADS * DHEAD)[:, perm]
    colh = jnp.arange(16) % NHEADS
    rowh = jcol % NHEADS
    rowd = jcol // NHEADS
    sel = rowh[:, None] == colh[None, :]
    Ms = jnp.where(sel, A[colh[None, :], rowd[:, None]], 0.0).astype(f32)
    Md = jnp.where(sel, A[colh[None, :], 16 + rowd[:, None]], 0.0).astype(f32)
    ab_row = Ab[colh][None, :].astype(f32)

    wh, extras = pl.pallas_call(
        _dense_body,
        out_shape=[jax.ShapeDtypeStruct((N, 128), jnp.bfloat16),
                   jax.ShapeDtypeStruct((N, 128), f32)],
    )(x, Wc, Wb_c, Ms, Md, ab_row)
    atab_s = extras[:, 0:16]
    atab_d = extras[:, 16:32]
    bnd = extras[0, 32:48]

    src = edge_index[0]
    dst = edge_index[1]

    mesh = plsc.VectorSubcoreMesh(core_axis_name="c", subcore_axis_name="s",
                                  num_cores=NC, num_subcores=NS)
    sc = pl.kernel(
        _sc_body,
        out_type=[jax.ShapeDtypeStruct((NC, N_PAD, 128), f32),
                  jax.ShapeDtypeStruct((NC, N_PAD, 16), f32)],
        mesh=mesh,
        compiler_params=pltpu.CompilerParams(use_tc_tiling_on_sc=False),
        scratch_types=[
            pltpu.VMEM_SHARED((N_PAD, 128), f32),
            pltpu.VMEM_SHARED((N_PAD, 16), f32),
            pltpu.VMEM((4, CHUNK), jnp.int32),
            pltpu.VMEM((4, CHUNK), jnp.int32),
            pltpu.VMEM((2 * CHUNK, 16), f32),
            pltpu.VMEM((2 * CHUNK, 16), f32),
            pltpu.VMEM((2 * CHUNK, 128), jnp.bfloat16),
            pltpu.VMEM((2 * CHUNK, 128), f32),
            pltpu.VMEM((16,), f32),
            pltpu.SemaphoreType.DMA,
            pltpu.SemaphoreType.DMA,
            pltpu.SemaphoreType.DMA,
        ],
    )
    num_parts, den_parts = sc(wh, atab_s, atab_d, bnd, src, dst)

    brep = (jnp.arange(16)[:, None] == (jcol[None, :] % NHEADS)).astype(f32)
    pj = (jcol % DHEAD) * NHEADS + jcol // DHEAD
    pmat = (jcol[:, None] == pj[None, :]).astype(f32)
    out = pl.pallas_call(
        _combine_body,
        out_shape=jax.ShapeDtypeStruct((N, 128), f32),
    )(num_parts, den_parts, brep, pmat)
    return out
